# baseline (speedup 1.0000x reference)
import functools

import jax
import jax.numpy as jnp
import numpy as np
from jax import lax
from jax.experimental import pallas as pl
from jax.experimental.pallas import tpu as pltpu
from jax.experimental.pallas import tpu_sc as plsc

_N = 10000
_E = 320000
_F = 128
_B = 64

_NCH = 512
_CROWS = 24
_NPAD = _NCH * _CROWS
_NW = 32
_CPW = _NCH // _NW
_EP = _E // _NW
_NG = _EP // 16
_CAP = 64
_LCAP = 832
_RB = 128

_DEG_HIST = np.array([0]*28 + [200,600,1200,1800,2400,1800,1200,600,200] + [0]*28, dtype=np.float64)
_bins = np.arange(_DEG_HIST.shape[0], dtype=np.float64)
_AVG_DEG_LOG = float((np.log(_bins + 1.0) * _DEG_HIST).sum() / _DEG_HIST.sum())

_PREC = jax.lax.Precision.HIGHEST


def _dot(a, b):
    return jnp.dot(a, b, precision=_PREC, preferred_element_type=jnp.float32)


_RBLK = 1024
_NBLK = _NPAD // _RBLK


def _t1_body(x_ref, w1_ref, b1_ref, w2_ref, a_ref, c_ref):
    xb = x_ref[...]
    a_ref[...] = _dot(xb, w1_ref[...]) + b1_ref[...]
    c_ref[...] = _dot(xb, w2_ref[...])


def _t1_call(x_pad, W1, b1r, W2):
    full = lambda shp: pl.BlockSpec(shp, lambda i: (0, 0))
    row = pl.BlockSpec((_RBLK, _F), lambda i: (i, 0))
    return pl.pallas_call(
        _t1_body,
        grid=(_NBLK,),
        in_specs=[row, full((_F, _F)), full((1, _F)), full((_F, _F))],
        out_specs=[row, row],
        out_shape=[jax.ShapeDtypeStruct((_NPAD, _F), jnp.float32)] * 2,
    )(x_pad, W1, b1r, W2)


def _combine_body(layer0, x_ref, a_ref, s1_ref, s2_ref, mn_ref, mx_ref,
                  cnt_ref, wp_ref, bp_ref, wl_ref, bl_ref, *rest):
    if layer0:
        w1n_ref, b1n_ref, w2n_ref, h_ref, an_ref, cn_ref = rest
    else:
        (h_ref,) = rest
    xb = x_ref[...]
    A = a_ref[...]
    S1 = s1_ref[...]
    S2 = s2_ref[...]
    cnt = cnt_ref[...]
    has = cnt > 0.0
    cntc = jnp.maximum(cnt, 1.0)
    inv = 1.0 / cntc
    zero = jnp.zeros_like(A)
    mean = jnp.where(has, A + S1 * inv, zero)
    mn = jnp.where(has, A + mn_ref[...], zero)
    mx = jnp.where(has, A + mx_ref[...], zero)
    mean2 = jnp.where(has, A * A + (2.0 * A * S1 + S2) * inv, zero)
    std = jnp.sqrt(jnp.maximum(mean2 - mean * mean, 0.0) + 1e-5)
    agg = jnp.concatenate([mean, mn, mx, std], axis=-1)
    lg = jnp.log(cntc + 1.0)
    amp = lg * (1.0 / _AVG_DEG_LOG)
    att = _AVG_DEG_LOG / lg
    wp = wp_ref[...]
    out = (_dot(xb, wp[0:_F])
           + _dot(agg, wp[_F:5 * _F])
           + _dot(agg * amp, wp[5 * _F:9 * _F])
           + _dot(agg * att, wp[9 * _F:13 * _F])
           + bp_ref[...])
    out = _dot(out, wl_ref[...]) + bl_ref[...]
    if layer0:
        h = jnp.maximum(out, 0.0)
        h_ref[...] = h
        an_ref[...] = _dot(h, w1n_ref[...]) + b1n_ref[...]
        cn_ref[...] = _dot(h, w2n_ref[...])
    else:
        h_ref[...] = out


def _combine_call(layer0, x_pad, A, S1, S2, Mn, Mx, cnt2d, W_post, b_postr,
                  W_lin, b_linr, extra):
    full = lambda shp: pl.BlockSpec(shp, lambda i: (0, 0))
    row = pl.BlockSpec((_RBLK, _F), lambda i: (i, 0))
    col = pl.BlockSpec((_RBLK, 1), lambda i: (i, 0))
    in_specs = [row, row, row, row, row, row, col,
                full((13 * _F, _F)), full((1, _F)), full((_F, _F)), full((1, _F))]
    args = [x_pad, A, S1, S2, Mn, Mx, cnt2d, W_post, b_postr, W_lin, b_linr]
    if layer0:
        in_specs += [full((_F, _F)), full((1, _F)), full((_F, _F))]
        args += list(extra)
        out_specs = [row, row, row]
        out_shape = [jax.ShapeDtypeStruct((_NPAD, _F), jnp.float32)] * 3
    else:
        out_specs = [row]
        out_shape = [jax.ShapeDtypeStruct((_NPAD, _F), jnp.float32)]
    return pl.pallas_call(
        functools.partial(_combine_body, layer0),
        grid=(_NBLK,),
        in_specs=in_specs,
        out_specs=out_specs,
        out_shape=out_shape,
    )(*args)


def _pool_body(h_ref, b_ref, wm1_ref, bm1_ref, wm2_ref, bm2_ref, o_ref,
               psum, pcnt):
    i = pl.program_id(0)

    @pl.when(i == 0)
    def _():
        psum[...] = jnp.zeros_like(psum)
        pcnt[...] = jnp.zeros_like(pcnt)

    hb = h_ref[...]
    bb = b_ref[...]
    onehot = (bb == lax.broadcasted_iota(jnp.int32, (_RBLK, _B), 1)).astype(jnp.float32)
    psum[...] += lax.dot_general(onehot, hb, (((0,), (0,)), ((), ())),
                                 precision=_PREC, preferred_element_type=jnp.float32)
    pcnt[...] += jnp.broadcast_to(jnp.sum(onehot, axis=0)[:, None], (_B, _F))

    @pl.when(i == _NBLK - 1)
    def _():
        pooled = psum[...] / jnp.maximum(pcnt[...], 1.0)
        hmid = jnp.maximum(_dot(pooled, wm1_ref[...]) + bm1_ref[...], 0.0)
        o_ref[...] = _dot(hmid, wm2_ref[...]) + bm2_ref[...]


def _pool_call(h2, batch_pad, W_mol1, b_mol1r, W_mol2, b_mol2r):
    full = lambda shp: pl.BlockSpec(shp, lambda i: (0, 0))
    row = pl.BlockSpec((_RBLK, _F), lambda i: (i, 0))
    col = pl.BlockSpec((_RBLK, 1), lambda i: (i, 0))
    return pl.pallas_call(
        _pool_body,
        grid=(_NBLK,),
        in_specs=[row, col, full((_F, _F)), full((1, _F)), full((_F, _F)),
                  full((1, _F))],
        out_specs=pl.BlockSpec((_B, _F), lambda i: (0, 0)),
        out_shape=jax.ShapeDtypeStruct((_B, _F), jnp.float32),
        scratch_shapes=[pltpu.VMEM((_B, _F), jnp.float32),
                        pltpu.VMEM((_B, _F), jnp.float32)],
    )(h2, batch_pad, W_mol1, b_mol1r, W_mol2, b_mol2r)


_MESH = plsc.VectorSubcoreMesh(core_axis_name="c", subcore_axis_name="s")


def _worker_id():
    return lax.axis_index("s") * 2 + lax.axis_index("c")


def _sread(ref1d, idx):
    return plsc.load_gather(ref1d, [jnp.broadcast_to(idx, (16,))])[0]


@functools.partial(
    pl.kernel,
    out_type=jax.ShapeDtypeStruct((_NW, _NCH, _CAP), jnp.int32),
    mesh=_MESH,
    compiler_params=pltpu.CompilerParams(needs_layout_passes=False,
                                         use_tc_tiling_on_sc=False),
    scratch_types=[
        pltpu.VMEM((_EP,), jnp.int32),
        pltpu.VMEM((_EP,), jnp.int32),
        pltpu.VMEM((_NCH, _CAP), jnp.int32),
        pltpu.VMEM((_NCH,), jnp.int32),
        pltpu.VMEM((16,), jnp.int32),
    ],
)
def _bucket_kernel(src_hbm, dst_hbm, bout, sbuf, dbuf, flat, curs, kscr):
    w = _worker_id()
    base = w * _EP
    pltpu.sync_copy(src_hbm.at[pl.ds(base, _EP)], sbuf)
    pltpu.sync_copy(dst_hbm.at[pl.ds(base, _EP)], dbuf)
    zeros16 = jnp.zeros((16,), jnp.int32)
    ones16 = jnp.ones((16,), jnp.int32)
    neg16 = jnp.full((16,), -1, jnp.int32)
    iota = lax.iota(jnp.int32, 16)
    for i in range(_NCH // 16):
        curs[pl.ds(i * 16, 16)] = zeros16

    def fill_row(rr, carry):
        for k in range(_CAP // 16):
            flat[rr, pl.ds(k * 16, 16)] = neg16
        return carry

    lax.fori_loop(0, _NCH, fill_row, 0)

    def body(g, carry):
        dvec = dbuf[pl.ds(g * 16, 16)]
        svec = sbuf[pl.ds(g * 16, 16)]
        cid = jnp.bitwise_and(dvec, _NCH - 1)
        dl = jnp.right_shift(dvec, 9)
        packed = svec * 32 + dl
        skey, sval = plsc.sort_key_val(cid, packed)
        kscr[pl.ds(0, 16)] = skey
        prev = plsc.load_gather(kscr, [jnp.maximum(iota - 1, 0)])
        boundary = jnp.logical_or(iota == 0, skey != prev)
        segstart = plsc.cummax(jnp.where(boundary, iota, 0))
        rank = iota - segstart
        cur = plsc.load_gather(curs, [skey])
        inner = jnp.minimum(cur + rank, _CAP - 1)
        plsc.store_scatter(flat, [skey, inner], sval)
        plsc.addupdate_scatter(curs, [skey], ones16)
        return carry

    lax.fori_loop(0, _NG, body, 0)
    pltpu.sync_copy(flat, bout.at[w])


@functools.partial(
    pl.kernel,
    out_type=[jax.ShapeDtypeStruct((_CROWS, _NCH, _F), jnp.float32),
              jax.ShapeDtypeStruct((_CROWS, _NCH, _F), jnp.float32),
              jax.ShapeDtypeStruct((_CROWS, _NCH, _F), jnp.float32),
              jax.ShapeDtypeStruct((_CROWS, _NCH, _F), jnp.float32),
              jax.ShapeDtypeStruct((_NCH, _CROWS), jnp.float32)],
    mesh=_MESH,
    compiler_params=pltpu.CompilerParams(needs_layout_passes=False,
                                         use_tc_tiling_on_sc=False),
    scratch_types=[
        pltpu.VMEM((32, _F), jnp.float32),
        pltpu.VMEM((32, _F), jnp.float32),
        pltpu.VMEM((32, _F), jnp.float32),
        pltpu.VMEM((32, _F), jnp.float32),
        pltpu.VMEM((32,), jnp.int32),
        pltpu.VMEM((32,), jnp.float32),
        pltpu.VMEM((32,), jnp.float32),
        pltpu.VMEM((32,), jnp.int32),
        pltpu.VMEM((32,), jnp.int32),
        pltpu.VMEM((_NW, _CAP), jnp.int32),
        pltpu.VMEM((_LCAP,), jnp.int32),
        pltpu.VMEM((_LCAP,), jnp.int32),
        pltpu.VMEM((_LCAP,), jnp.int32),
        pltpu.VMEM((_LCAP,), jnp.int32),
        pltpu.VMEM((_LCAP, _F), jnp.float32),
        pltpu.VMEM((16,), jnp.int32),
        pltpu.SemaphoreType.DMA,
    ],
)
def _accum_kernel(c_hbm, bkt_hbm, s1o, s2o, mno, mxo, cnto,
                  s1, s2, mn, mx, cli, clf, codd, segb, scur, lists, sidx,
                  dlv, ssidx, ssdl, rows, kscr, sem):
    w = _worker_id()
    zeros16f = jnp.zeros((16,), jnp.float32)
    big16 = jnp.full((16,), 3e38, jnp.float32)
    zi16 = jnp.zeros((16,), jnp.int32)
    hi16 = jnp.full((16,), 31, jnp.int32)
    ones16 = jnp.ones((16,), jnp.int32)
    iota = lax.iota(jnp.int32, 16)

    def chunk_body(chunk_i, carry0):
        c = w * _CPW + chunk_i

        def zfill(k, carry):
            sl = pl.ds(k * 16, 16)
            sidx[sl] = zi16
            dlv[sl] = hi16
            ssidx[sl] = zi16
            ssdl[sl] = hi16
            return carry

        lax.fori_loop(0, _LCAP // 16, zfill, 0)
        cli[pl.ds(0, 16)] = zi16
        cli[pl.ds(16, 16)] = zi16

        def init_row(rr, carry):
            for j in range(8):
                sl = pl.ds(j * 16, 16)
                s1[rr, sl] = zeros16f
                s2[rr, sl] = zeros16f
                mn[rr, sl] = big16
                mx[rr, sl] = -big16
            return carry

        lax.fori_loop(0, 32, init_row, 0)

        descs = []
        for t in range(_NW):
            descs.append(pltpu.async_copy(bkt_hbm.at[t, c, :], lists.at[t],
                                          sem))
        for d in descs:
            d.wait()

        def compact_t(t, cur):
            def g_body(gi, cur2):
                v = lists[t, pl.ds(gi * 16, 16)]
                msk = v >= 0
                dl = jnp.bitwise_and(v, 31)
                plsc.store_compressed(sidx.at[pl.ds(cur2, 16)],
                                      jnp.right_shift(v, 5), mask=msk)
                plsc.store_compressed(dlv.at[pl.ds(cur2, 16)], dl, mask=msk)
                plsc.addupdate_scatter(cli, [dl], ones16, mask=msk)
                return cur2 + jnp.sum(msk.astype(jnp.int32))

            return lax.fori_loop(0, _CAP // 16, g_body, cur)

        m_total = lax.fori_loop(0, _NW, compact_t, 0)

        c0 = cli[pl.ds(0, 16)]
        c1 = cli[pl.ds(16, 16)]
        o0 = jnp.bitwise_and(c0, 1)
        o1 = jnp.bitwise_and(c1, 1)
        codd[pl.ds(0, 16)] = o0.astype(jnp.float32)
        codd[pl.ds(16, 16)] = o1.astype(jnp.float32)
        a0 = c0 + o0
        a1 = c1 + o1
        s0 = plsc.cumsum(a0)
        s1c = plsc.cumsum(a1) + s0[15]
        e0 = s0 - a0
        e1 = s1c - a1
        segb[pl.ds(0, 16)] = e0
        segb[pl.ds(16, 16)] = e1
        scur[pl.ds(0, 16)] = e0
        scur[pl.ds(16, 16)] = e1
        total2 = jnp.minimum(s1c[15], (_LCAP // _RB) * _RB)

        ngrp = (m_total + 15) // 16

        def sort_g(gi, carry):
            dl = dlv[pl.ds(gi * 16, 16)]
            sv = sidx[pl.ds(gi * 16, 16)]
            skey, sval = plsc.sort_key_val(dl, sv)
            kscr[pl.ds(0, 16)] = skey
            prev = plsc.load_gather(kscr, [jnp.maximum(iota - 1, 0)])
            boundary = jnp.logical_or(iota == 0, skey != prev)
            segstart = plsc.cummax(jnp.where(boundary, iota, 0))
            rank = iota - segstart
            cur = plsc.load_gather(scur, [skey])
            plsc.store_scatter(ssidx, [cur + rank], sval)
            plsc.store_scatter(ssdl, [cur + rank], skey)
            plsc.addupdate_scatter(scur, [skey], ones16)
            return carry

        lax.fori_loop(0, ngrp, sort_g, 0)

        for off, ov in ((0, o0), (16, o1)):
            basev = segb[pl.ds(off, 16)]
            kev = cli[pl.ds(off, 16)]
            padmask = ov == 1
            pad_val = plsc.load_gather(ssidx, [basev])
            plsc.store_scatter(ssidx, [basev + kev], pad_val, mask=padmask)

        nb = (total2 + (_RB - 1)) // _RB

        def fire_b(b, carry):
            pltpu.async_copy(c_hbm.at[ssidx.at[pl.ds(b * _RB, _RB)]],
                             rows.at[pl.ds(b * _RB, _RB), :], sem)
            return carry

        lax.fori_loop(0, nb, fire_b, 0)

        def drain_b(b, carry):
            pltpu.make_async_copy(c_hbm.at[ssidx.at[pl.ds(0, _RB)]],
                                  rows.at[pl.ds(0, _RB), :], sem).wait()
            return carry

        lax.fori_loop(0, nb, drain_b, 0)

        ngp = (total2 + 31) // 32

        def pair_g(gp, carry):
            eb = gp * 32
            dla = ssdl[pl.ds(eb, 16)]
            dlb = ssdl[pl.ds(eb + 16, 16)]
            for half, dvec in ((0, dla), (1, dlb)):
                for i in range(8):
                    d = dvec[2 * i]
                    q = eb + half * 16 + 2 * i
                    for j in range(8):
                        sl = pl.ds(j * 16, 16)
                        g0 = rows[q, sl]
                        g1 = rows[q + 1, sl]
                        plsc.addupdate(s1.at[d, sl], g0 + g1)
                        plsc.addupdate(s2.at[d, sl], g0 * g0 + g1 * g1)
                        mn[d, sl] = jnp.minimum(mn[d, sl],
                                                jnp.minimum(g0, g1))
                        mx[d, sl] = jnp.maximum(mx[d, sl],
                                                jnp.maximum(g0, g1))
            return carry

        lax.fori_loop(0, ngp, pair_g, 0)

        def fix_body(rr, carry):
            pc = _sread(codd, rr)
            b2 = _sread(segb, rr)
            pcv = jnp.broadcast_to(pc, (16,))
            for j in range(8):
                sl = pl.ds(j * 16, 16)
                gf = rows[b2, sl]
                plsc.addupdate(s1.at[rr, sl], -(pcv * gf))
                plsc.addupdate(s2.at[rr, sl], -(pcv * gf * gf))
            return carry

        lax.fori_loop(0, _CROWS, fix_body, 0)

        clf[pl.ds(0, 16)] = cli[pl.ds(0, 16)].astype(jnp.float32)
        clf[pl.ds(16, 16)] = cli[pl.ds(16, 16)].astype(jnp.float32)

        pltpu.sync_copy(s1.at[pl.ds(0, _CROWS), :], s1o.at[:, c, :])
        pltpu.sync_copy(s2.at[pl.ds(0, _CROWS), :], s2o.at[:, c, :])
        pltpu.sync_copy(mn.at[pl.ds(0, _CROWS), :], mno.at[:, c, :])
        pltpu.sync_copy(mx.at[pl.ds(0, _CROWS), :], mxo.at[:, c, :])
        pltpu.sync_copy(clf.at[pl.ds(0, _CROWS)], cnto.at[c])
        return carry0

    lax.fori_loop(0, _CPW, chunk_body, 0)


def kernel(x, edge_index, batch, W_pre_0, b_pre_0, W_post_0, b_post_0,
           W_lin_0, b_lin_0, W_pre_1, b_pre_1, W_post_1, b_post_1, W_lin_1,
           b_lin_1, W_mol1, b_mol1, W_mol2, b_mol2):
    src = edge_index[0]
    dst = edge_index[1]
    x_pad = jnp.pad(x, ((0, _NPAD - _N), (0, 0)))
    batch_pad = jnp.pad(batch, (0, _NPAD - _N), constant_values=_B).reshape(_NPAD, 1)
    W1_0, W2_0 = W_pre_0[:_F], W_pre_0[_F:]
    W1_1, W2_1 = W_pre_1[:_F], W_pre_1[_F:]
    r = lambda b: b.reshape(1, _F)

    A0, C0 = _t1_call(x_pad, W1_0, r(b_pre_0), W2_0)
    buckets = _bucket_kernel(src, dst)

    S1a, S2a, Mna, Mxa, cnta = _accum_kernel(C0, buckets)
    cnt2d = cnta.T.reshape(_NPAD, 1)
    h, A1, C1 = _combine_call(
        True, x_pad, A0, S1a.reshape(_NPAD, _F), S2a.reshape(_NPAD, _F),
        Mna.reshape(_NPAD, _F), Mxa.reshape(_NPAD, _F), cnt2d, W_post_0,
        r(b_post_0), W_lin_0, r(b_lin_0), (W1_1, r(b_pre_1), W2_1))

    S1b, S2b, Mnb, Mxb, _cntb = _accum_kernel(C1, buckets)
    (h2,) = _combine_call(
        False, h, A1, S1b.reshape(_NPAD, _F), S2b.reshape(_NPAD, _F),
        Mnb.reshape(_NPAD, _F), Mxb.reshape(_NPAD, _F), cnt2d, W_post_1,
        r(b_post_1), W_lin_1, r(b_lin_1), ())

    return _pool_call(h2, batch_pad, W_mol1, r(b_mol1), W_mol2, r(b_mol2))


# speedup vs baseline: 1.6349x; 1.6349x over previous
import functools

import jax
import jax.numpy as jnp
import numpy as np
from jax import lax
from jax.experimental import pallas as pl
from jax.experimental.pallas import tpu as pltpu
from jax.experimental.pallas import tpu_sc as plsc

_N = 10000
_E = 320000
_F = 128
_B = 64

_NCH = 128
_CROWS = 80
_NPAD = _NCH * _CROWS
_NW = 32
_CPW = _NCH // _NW
_EP = _E // _NW
_NG = _EP // 16
_CAP = 256
_LCAP = 2816
_RB = 128
_SPARE = _CROWS - 1

_DEG_HIST = np.array([0]*28 + [200,600,1200,1800,2400,1800,1200,600,200] + [0]*28, dtype=np.float64)
_bins = np.arange(_DEG_HIST.shape[0], dtype=np.float64)
_AVG_DEG_LOG = float((np.log(_bins + 1.0) * _DEG_HIST).sum() / _DEG_HIST.sum())

_PREC = jax.lax.Precision.HIGHEST


def _dot(a, b):
    return jnp.dot(a, b, precision=_PREC, preferred_element_type=jnp.float32)


_RBLK = 1024
_NBLK = _NPAD // _RBLK


def _t1_body(x_ref, w1_ref, b1_ref, w2_ref, a_ref, c_ref):
    xb = x_ref[...]
    a_ref[...] = _dot(xb, w1_ref[...]) + b1_ref[...]
    c_ref[...] = _dot(xb, w2_ref[...])


def _t1_call(x_pad, W1, b1r, W2):
    full = lambda shp: pl.BlockSpec(shp, lambda i: (0, 0))
    row = pl.BlockSpec((_RBLK, _F), lambda i: (i, 0))
    return pl.pallas_call(
        _t1_body,
        grid=(_NBLK,),
        in_specs=[row, full((_F, _F)), full((1, _F)), full((_F, _F))],
        out_specs=[row, row],
        out_shape=[jax.ShapeDtypeStruct((_NPAD, _F), jnp.float32)] * 2,
    )(x_pad, W1, b1r, W2)


def _combine_body(layer0, x_ref, a_ref, s1_ref, s2_ref, mn_ref, mx_ref,
                  cnt_ref, wp_ref, bp_ref, wl_ref, bl_ref, *rest):
    if layer0:
        w1n_ref, b1n_ref, w2n_ref, h_ref, an_ref, cn_ref = rest
    else:
        (h_ref,) = rest
    xb = x_ref[...]
    A = a_ref[...]
    S1 = s1_ref[...]
    S2 = s2_ref[...]
    cnt = cnt_ref[...]
    has = cnt > 0.0
    cntc = jnp.maximum(cnt, 1.0)
    inv = 1.0 / cntc
    zero = jnp.zeros_like(A)
    mean = jnp.where(has, A + S1 * inv, zero)
    mn = jnp.where(has, A + mn_ref[...], zero)
    mx = jnp.where(has, A + mx_ref[...], zero)
    mean2 = jnp.where(has, A * A + (2.0 * A * S1 + S2) * inv, zero)
    std = jnp.sqrt(jnp.maximum(mean2 - mean * mean, 0.0) + 1e-5)
    agg = jnp.concatenate([mean, mn, mx, std], axis=-1)
    lg = jnp.log(cntc + 1.0)
    amp = lg * (1.0 / _AVG_DEG_LOG)
    att = _AVG_DEG_LOG / lg
    wp = wp_ref[...]
    out = (_dot(xb, wp[0:_F])
           + _dot(agg, wp[_F:5 * _F])
           + _dot(agg * amp, wp[5 * _F:9 * _F])
           + _dot(agg * att, wp[9 * _F:13 * _F])
           + bp_ref[...])
    out = _dot(out, wl_ref[...]) + bl_ref[...]
    if layer0:
        h = jnp.maximum(out, 0.0)
        h_ref[...] = h
        an_ref[...] = _dot(h, w1n_ref[...]) + b1n_ref[...]
        cn_ref[...] = _dot(h, w2n_ref[...])
    else:
        h_ref[...] = out


def _combine_call(layer0, x_pad, A, S1, S2, Mn, Mx, cnt2d, W_post, b_postr,
                  W_lin, b_linr, extra):
    full = lambda shp: pl.BlockSpec(shp, lambda i: (0, 0))
    row = pl.BlockSpec((_RBLK, _F), lambda i: (i, 0))
    col = pl.BlockSpec((_RBLK, 1), lambda i: (i, 0))
    in_specs = [row, row, row, row, row, row, col,
                full((13 * _F, _F)), full((1, _F)), full((_F, _F)), full((1, _F))]
    args = [x_pad, A, S1, S2, Mn, Mx, cnt2d, W_post, b_postr, W_lin, b_linr]
    if layer0:
        in_specs += [full((_F, _F)), full((1, _F)), full((_F, _F))]
        args += list(extra)
        out_specs = [row, row, row]
        out_shape = [jax.ShapeDtypeStruct((_NPAD, _F), jnp.float32)] * 3
    else:
        out_specs = [row]
        out_shape = [jax.ShapeDtypeStruct((_NPAD, _F), jnp.float32)]
    return pl.pallas_call(
        functools.partial(_combine_body, layer0),
        grid=(_NBLK,),
        in_specs=in_specs,
        out_specs=out_specs,
        out_shape=out_shape,
    )(*args)


def _pool_body(h_ref, b_ref, wm1_ref, bm1_ref, wm2_ref, bm2_ref, o_ref,
               psum, pcnt):
    i = pl.program_id(0)

    @pl.when(i == 0)
    def _():
        psum[...] = jnp.zeros_like(psum)
        pcnt[...] = jnp.zeros_like(pcnt)

    hb = h_ref[...]
    bb = b_ref[...]
    onehot = (bb == lax.broadcasted_iota(jnp.int32, (_RBLK, _B), 1)).astype(jnp.float32)
    psum[...] += lax.dot_general(onehot, hb, (((0,), (0,)), ((), ())),
                                 precision=_PREC, preferred_element_type=jnp.float32)
    pcnt[...] += jnp.broadcast_to(jnp.sum(onehot, axis=0)[:, None], (_B, _F))

    @pl.when(i == _NBLK - 1)
    def _():
        pooled = psum[...] / jnp.maximum(pcnt[...], 1.0)
        hmid = jnp.maximum(_dot(pooled, wm1_ref[...]) + bm1_ref[...], 0.0)
        o_ref[...] = _dot(hmid, wm2_ref[...]) + bm2_ref[...]


def _pool_call(h2, batch_pad, W_mol1, b_mol1r, W_mol2, b_mol2r):
    full = lambda shp: pl.BlockSpec(shp, lambda i: (0, 0))
    row = pl.BlockSpec((_RBLK, _F), lambda i: (i, 0))
    col = pl.BlockSpec((_RBLK, 1), lambda i: (i, 0))
    return pl.pallas_call(
        _pool_body,
        grid=(_NBLK,),
        in_specs=[row, col, full((_F, _F)), full((1, _F)), full((_F, _F)),
                  full((1, _F))],
        out_specs=pl.BlockSpec((_B, _F), lambda i: (0, 0)),
        out_shape=jax.ShapeDtypeStruct((_B, _F), jnp.float32),
        scratch_shapes=[pltpu.VMEM((_B, _F), jnp.float32),
                        pltpu.VMEM((_B, _F), jnp.float32)],
    )(h2, batch_pad, W_mol1, b_mol1r, W_mol2, b_mol2r)


_MESH = plsc.VectorSubcoreMesh(core_axis_name="c", subcore_axis_name="s")


def _worker_id():
    return lax.axis_index("s") * 2 + lax.axis_index("c")


def _sread(ref1d, idx):
    return plsc.load_gather(ref1d, [jnp.broadcast_to(idx, (16,))])[0]


@functools.partial(
    pl.kernel,
    out_type=[jax.ShapeDtypeStruct((_NW, _NCH * _CAP), jnp.int32),
              jax.ShapeDtypeStruct((_NW * _NCH,), jnp.int32)],
    mesh=_MESH,
    compiler_params=pltpu.CompilerParams(needs_layout_passes=False),
    scratch_types=[
        pltpu.VMEM((_EP,), jnp.int32),
        pltpu.VMEM((_EP,), jnp.int32),
        pltpu.VMEM((_NCH * _CAP,), jnp.int32),
        pltpu.VMEM((_NCH,), jnp.int32),
        pltpu.VMEM((16,), jnp.int32),
    ],
)
def _bucket_kernel(src_hbm, dst_hbm, bout, cout, sbuf, dbuf, flat, curs, kscr):
    w = _worker_id()
    base = w * _EP
    pltpu.sync_copy(src_hbm.at[pl.ds(base, _EP)], sbuf)
    pltpu.sync_copy(dst_hbm.at[pl.ds(base, _EP)], dbuf)
    zeros16 = jnp.zeros((16,), jnp.int32)
    ones16 = jnp.ones((16,), jnp.int32)
    iota = lax.iota(jnp.int32, 16)
    for i in range(_NCH // 16):
        curs[pl.ds(i * 16, 16)] = zeros16

    def body(g, carry):
        dvec = dbuf[pl.ds(g * 16, 16)]
        svec = sbuf[pl.ds(g * 16, 16)]
        cid = jnp.bitwise_and(dvec, _NCH - 1)
        dl = jnp.right_shift(dvec, 7)
        packed = svec * 128 + dl
        skey, sval = plsc.sort_key_val(cid, packed)
        kscr[pl.ds(0, 16)] = skey
        prev = plsc.load_gather(kscr, [jnp.maximum(iota - 1, 0)])
        boundary = jnp.logical_or(iota == 0, skey != prev)
        segstart = plsc.cummax(jnp.where(boundary, iota, 0))
        rank = iota - segstart
        cur = plsc.load_gather(curs, [skey])
        pos = jnp.minimum(skey * _CAP + cur + rank, skey * _CAP + (_CAP - 1))
        plsc.store_scatter(flat, [pos], sval)
        plsc.addupdate_scatter(curs, [skey], ones16)
        return carry

    lax.fori_loop(0, _NG, body, 0)
    pltpu.sync_copy(flat, bout.at[w])
    pltpu.sync_copy(curs, cout.at[pl.ds(w * _NCH, _NCH)])


@functools.partial(
    pl.kernel,
    out_type=[jax.ShapeDtypeStruct((_CROWS, _NCH, _F), jnp.float32),
              jax.ShapeDtypeStruct((_CROWS, _NCH, _F), jnp.float32),
              jax.ShapeDtypeStruct((_CROWS, _NCH, _F), jnp.float32),
              jax.ShapeDtypeStruct((_CROWS, _NCH, _F), jnp.float32),
              jax.ShapeDtypeStruct((_NCH, _CROWS), jnp.float32)],
    mesh=_MESH,
    compiler_params=pltpu.CompilerParams(needs_layout_passes=False),
    scratch_types=[
        pltpu.VMEM((_CROWS, _F), jnp.float32),
        pltpu.VMEM((_CROWS, _F), jnp.float32),
        pltpu.VMEM((_CROWS, _F), jnp.float32),
        pltpu.VMEM((_CROWS, _F), jnp.float32),
        pltpu.VMEM((_CROWS,), jnp.int32),
        pltpu.VMEM((_CROWS,), jnp.float32),
        pltpu.VMEM((_NW * _CAP,), jnp.int32),
        pltpu.VMEM((_LCAP,), jnp.int32),
        pltpu.VMEM((_LCAP,), jnp.int32),
        pltpu.VMEM((2 * _RB, _F), jnp.float32),
        pltpu.VMEM((_NW * _NCH,), jnp.int32),
        pltpu.SemaphoreType.DMA,
    ],
)
def _accum_kernel(c_hbm, bkt_hbm, cnt_hbm, s1o, s2o, mno, mxo, cnto,
                  s1, s2, mn, mx, cli, clf, lists, sidx, dlv, rows, cstage,
                  sem):
    w = _worker_id()
    pltpu.sync_copy(cnt_hbm, cstage)
    zeros16 = jnp.zeros((16,), jnp.float32)
    big16 = jnp.full((16,), 3e38, jnp.float32)
    zi16 = jnp.zeros((16,), jnp.int32)
    spare16 = jnp.full((16,), _SPARE, jnp.int32)
    ones16 = jnp.ones((16,), jnp.int32)
    iota = lax.iota(jnp.int32, 16)

    def chunk_body(chunk_i, carry0):
        c = w * _CPW + chunk_i

        def init_row(rr, carry):
            for j in range(8):
                sl = pl.ds(j * 16, 16)
                s1[rr, sl] = zeros16
                s2[rr, sl] = zeros16
                mn[rr, sl] = big16
                mx[rr, sl] = -big16
            return carry

        lax.fori_loop(0, _CROWS, init_row, 0)

        def zfill(k, carry):
            sl = pl.ds(k * 16, 16)
            sidx[sl] = zi16
            dlv[sl] = spare16
            return carry

        lax.fori_loop(0, _LCAP // 16, zfill, 0)
        for k in range(_CROWS // 16):
            cli[pl.ds(k * 16, 16)] = zi16

        descs = []
        for t in range(_NW):
            descs.append(pltpu.async_copy(
                bkt_hbm.at[t, pl.ds(c * _CAP, _CAP)],
                lists.at[pl.ds(t * _CAP, _CAP)], sem))
        for d in descs:
            d.wait()

        def compact_t(t, cur):
            nt = jnp.minimum(_sread(cstage, t * _NCH + c), _CAP)
            ng = (nt + 15) // 16

            def g_body(gi, cur2):
                v = lists[pl.ds(t * _CAP + gi * 16, 16)]
                msk = (gi * 16 + iota) < nt
                dl = jnp.bitwise_and(v, _NCH - 1)
                plsc.store_compressed(sidx.at[pl.ds(cur2, 16)],
                                      jnp.right_shift(v, 7), mask=msk)
                plsc.store_compressed(dlv.at[pl.ds(cur2, 16)], dl, mask=msk)
                plsc.addupdate_scatter(cli, [dl], ones16, mask=msk)
                return cur2 + jnp.sum(msk.astype(jnp.int32))

            return lax.fori_loop(0, ng, g_body, cur)

        m_total = lax.fori_loop(0, _NW, compact_t, 0)
        nb = (m_total + (_RB - 1)) // _RB

        def fire(b, par):
            pltpu.async_copy(
                c_hbm.at[sidx.at[pl.ds(b * _RB, _RB)]],
                rows.at[pl.ds(par * _RB, _RB), :], sem)

        @pl.when(nb > 0)
        def _():
            fire(0, 0)

        def batch_body(b, carry):
            par = jnp.bitwise_and(b, 1)

            @pl.when(b + 1 < nb)
            def _():
                fire(b + 1, 1 - par)

            pltpu.make_async_copy(
                c_hbm.at[sidx.at[pl.ds(b * _RB, _RB)]],
                rows.at[pl.ds(par * _RB, _RB), :], sem).wait()
            rbase = par * _RB

            def group_body(gi, carry2):
                ebase = gi * 16
                dvec = dlv[pl.ds(b * _RB + ebase, 16)]
                for i in range(16):
                    d = dvec[i]
                    r = rbase + ebase + i
                    for j in range(8):
                        sl = pl.ds(j * 16, 16)
                        g = rows[r, sl]
                        plsc.addupdate(s1.at[d, sl], g)
                        plsc.addupdate(s2.at[d, sl], g * g)
                        mn[d, sl] = jnp.minimum(mn[d, sl], g)
                        mx[d, sl] = jnp.maximum(mx[d, sl], g)
                return carry2

            lax.fori_loop(0, _RB // 16, group_body, 0)
            return carry

        lax.fori_loop(0, nb, batch_body, 0)

        for k in range(_CROWS // 16):
            sl = pl.ds(k * 16, 16)
            clf[sl] = cli[sl].astype(jnp.float32)

        pltpu.sync_copy(s1, s1o.at[:, c, :])
        pltpu.sync_copy(s2, s2o.at[:, c, :])
        pltpu.sync_copy(mn, mno.at[:, c, :])
        pltpu.sync_copy(mx, mxo.at[:, c, :])
        pltpu.sync_copy(clf, cnto.at[c])
        return carry0

    lax.fori_loop(0, _CPW, chunk_body, 0)


def kernel(x, edge_index, batch, W_pre_0, b_pre_0, W_post_0, b_post_0,
           W_lin_0, b_lin_0, W_pre_1, b_pre_1, W_post_1, b_post_1, W_lin_1,
           b_lin_1, W_mol1, b_mol1, W_mol2, b_mol2):
    src = edge_index[0]
    dst = edge_index[1]
    x_pad = jnp.pad(x, ((0, _NPAD - _N), (0, 0)))
    batch_pad = jnp.pad(batch, (0, _NPAD - _N), constant_values=_B).reshape(_NPAD, 1)
    W1_0, W2_0 = W_pre_0[:_F], W_pre_0[_F:]
    W1_1, W2_1 = W_pre_1[:_F], W_pre_1[_F:]
    r = lambda b: b.reshape(1, _F)

    A0, C0 = _t1_call(x_pad, W1_0, r(b_pre_0), W2_0)
    buckets, counts = _bucket_kernel(src, dst)

    S1a, S2a, Mna, Mxa, cnta = _accum_kernel(C0, buckets, counts)
    cnt2d = cnta.T.reshape(_NPAD, 1)
    h, A1, C1 = _combine_call(
        True, x_pad, A0, S1a.reshape(_NPAD, _F), S2a.reshape(_NPAD, _F),
        Mna.reshape(_NPAD, _F), Mxa.reshape(_NPAD, _F), cnt2d, W_post_0,
        r(b_post_0), W_lin_0, r(b_lin_0), (W1_1, r(b_pre_1), W2_1))

    S1b, S2b, Mnb, Mxb, _cntb = _accum_kernel(C1, buckets, counts)
    (h2,) = _combine_call(
        False, h, A1, S1b.reshape(_NPAD, _F), S2b.reshape(_NPAD, _F),
        Mnb.reshape(_NPAD, _F), Mxb.reshape(_NPAD, _F), cnt2d, W_post_1,
        r(b_post_1), W_lin_1, r(b_lin_1), ())

    return _pool_call(h2, batch_pad, W_mol1, r(b_mol1), W_mol2, r(b_mol2))


# speedup vs baseline: 1.6618x; 1.0165x over previous
import functools

import jax
import jax.numpy as jnp
import numpy as np
from jax import lax
from jax.experimental import pallas as pl
from jax.experimental.pallas import tpu as pltpu
from jax.experimental.pallas import tpu_sc as plsc

_N = 10000
_E = 320000
_F = 128
_B = 64

_NCH = 128
_CROWS = 80
_NPAD = _NCH * _CROWS
_NW = 32
_CPW = _NCH // _NW
_EP = _E // _NW
_NG = _EP // 16
_CAP = 256
_LCAP = 2816
_RB = 128
_SPARE = _CROWS - 1

_DEG_HIST = np.array([0]*28 + [200,600,1200,1800,2400,1800,1200,600,200] + [0]*28, dtype=np.float64)
_bins = np.arange(_DEG_HIST.shape[0], dtype=np.float64)
_AVG_DEG_LOG = float((np.log(_bins + 1.0) * _DEG_HIST).sum() / _DEG_HIST.sum())

_PREC = jax.lax.Precision.HIGHEST


def _dot(a, b):
    return jnp.dot(a, b, precision=_PREC, preferred_element_type=jnp.float32)


_RBLK = 1024
_NBLK = _NPAD // _RBLK


def _t1_body(x_ref, w1_ref, b1_ref, w2_ref, a_ref, c_ref):
    xb = x_ref[...]
    a_ref[...] = _dot(xb, w1_ref[...]) + b1_ref[...]
    c_ref[...] = _dot(xb, w2_ref[...])


def _t1_call(x_pad, W1, b1r, W2):
    full = lambda shp: pl.BlockSpec(shp, lambda i: (0, 0))
    row = pl.BlockSpec((_RBLK, _F), lambda i: (i, 0))
    return pl.pallas_call(
        _t1_body,
        grid=(_NBLK,),
        in_specs=[row, full((_F, _F)), full((1, _F)), full((_F, _F))],
        out_specs=[row, row],
        out_shape=[jax.ShapeDtypeStruct((_NPAD, _F), jnp.float32)] * 2,
    )(x_pad, W1, b1r, W2)


def _combine_body(layer0, x_ref, a_ref, s1_ref, s2_ref, mn_ref, mx_ref,
                  cnt_ref, wp_ref, bp_ref, wl_ref, bl_ref, *rest):
    if layer0:
        w1n_ref, b1n_ref, w2n_ref, h_ref, an_ref, cn_ref = rest
    else:
        (h_ref,) = rest
    xb = x_ref[...]
    A = a_ref[...]
    S1 = s1_ref[...]
    S2 = s2_ref[...]
    cnt = cnt_ref[...]
    has = cnt > 0.0
    cntc = jnp.maximum(cnt, 1.0)
    inv = 1.0 / cntc
    zero = jnp.zeros_like(A)
    mean = jnp.where(has, A + S1 * inv, zero)
    mn = jnp.where(has, A + mn_ref[...], zero)
    mx = jnp.where(has, A + mx_ref[...], zero)
    mean2 = jnp.where(has, A * A + (2.0 * A * S1 + S2) * inv, zero)
    std = jnp.sqrt(jnp.maximum(mean2 - mean * mean, 0.0) + 1e-5)
    agg = jnp.concatenate([mean, mn, mx, std], axis=-1)
    lg = jnp.log(cntc + 1.0)
    amp = lg * (1.0 / _AVG_DEG_LOG)
    att = _AVG_DEG_LOG / lg
    wp = wp_ref[...]
    out = (_dot(xb, wp[0:_F])
           + _dot(agg, wp[_F:5 * _F])
           + _dot(agg * amp, wp[5 * _F:9 * _F])
           + _dot(agg * att, wp[9 * _F:13 * _F])
           + bp_ref[...])
    out = _dot(out, wl_ref[...]) + bl_ref[...]
    if layer0:
        h = jnp.maximum(out, 0.0)
        h_ref[...] = h
        an_ref[...] = _dot(h, w1n_ref[...]) + b1n_ref[...]
        cn_ref[...] = _dot(h, w2n_ref[...])
    else:
        h_ref[...] = out


def _combine_call(layer0, x_pad, A, S1, S2, Mn, Mx, cnt2d, W_post, b_postr,
                  W_lin, b_linr, extra):
    full = lambda shp: pl.BlockSpec(shp, lambda i: (0, 0))
    row = pl.BlockSpec((_RBLK, _F), lambda i: (i, 0))
    col = pl.BlockSpec((_RBLK, 1), lambda i: (i, 0))
    in_specs = [row, row, row, row, row, row, col,
                full((13 * _F, _F)), full((1, _F)), full((_F, _F)), full((1, _F))]
    args = [x_pad, A, S1, S2, Mn, Mx, cnt2d, W_post, b_postr, W_lin, b_linr]
    if layer0:
        in_specs += [full((_F, _F)), full((1, _F)), full((_F, _F))]
        args += list(extra)
        out_specs = [row, row, row]
        out_shape = [jax.ShapeDtypeStruct((_NPAD, _F), jnp.float32)] * 3
    else:
        out_specs = [row]
        out_shape = [jax.ShapeDtypeStruct((_NPAD, _F), jnp.float32)]
    return pl.pallas_call(
        functools.partial(_combine_body, layer0),
        grid=(_NBLK,),
        in_specs=in_specs,
        out_specs=out_specs,
        out_shape=out_shape,
    )(*args)


def _pool_body(h_ref, b_ref, wm1_ref, bm1_ref, wm2_ref, bm2_ref, o_ref,
               psum, pcnt):
    i = pl.program_id(0)

    @pl.when(i == 0)
    def _():
        psum[...] = jnp.zeros_like(psum)
        pcnt[...] = jnp.zeros_like(pcnt)

    hb = h_ref[...]
    bb = b_ref[...]
    onehot = (bb == lax.broadcasted_iota(jnp.int32, (_RBLK, _B), 1)).astype(jnp.float32)
    psum[...] += lax.dot_general(onehot, hb, (((0,), (0,)), ((), ())),
                                 precision=_PREC, preferred_element_type=jnp.float32)
    pcnt[...] += jnp.broadcast_to(jnp.sum(onehot, axis=0)[:, None], (_B, _F))

    @pl.when(i == _NBLK - 1)
    def _():
        pooled = psum[...] / jnp.maximum(pcnt[...], 1.0)
        hmid = jnp.maximum(_dot(pooled, wm1_ref[...]) + bm1_ref[...], 0.0)
        o_ref[...] = _dot(hmid, wm2_ref[...]) + bm2_ref[...]


def _pool_call(h2, batch_pad, W_mol1, b_mol1r, W_mol2, b_mol2r):
    full = lambda shp: pl.BlockSpec(shp, lambda i: (0, 0))
    row = pl.BlockSpec((_RBLK, _F), lambda i: (i, 0))
    col = pl.BlockSpec((_RBLK, 1), lambda i: (i, 0))
    return pl.pallas_call(
        _pool_body,
        grid=(_NBLK,),
        in_specs=[row, col, full((_F, _F)), full((1, _F)), full((_F, _F)),
                  full((1, _F))],
        out_specs=pl.BlockSpec((_B, _F), lambda i: (0, 0)),
        out_shape=jax.ShapeDtypeStruct((_B, _F), jnp.float32),
        scratch_shapes=[pltpu.VMEM((_B, _F), jnp.float32),
                        pltpu.VMEM((_B, _F), jnp.float32)],
    )(h2, batch_pad, W_mol1, b_mol1r, W_mol2, b_mol2r)


_MESH = plsc.VectorSubcoreMesh(core_axis_name="c", subcore_axis_name="s")


def _worker_id():
    return lax.axis_index("s") * 2 + lax.axis_index("c")


def _sread(ref1d, idx):
    return plsc.load_gather(ref1d, [jnp.broadcast_to(idx, (16,))])[0]


@functools.partial(
    pl.kernel,
    out_type=[jax.ShapeDtypeStruct((_NW, _NCH * _CAP), jnp.int32),
              jax.ShapeDtypeStruct((_NW * _NCH,), jnp.int32)],
    mesh=_MESH,
    compiler_params=pltpu.CompilerParams(needs_layout_passes=False),
    scratch_types=[
        pltpu.VMEM((_EP,), jnp.int32),
        pltpu.VMEM((_EP,), jnp.int32),
        pltpu.VMEM((_NCH * _CAP,), jnp.int32),
        pltpu.VMEM((_NCH,), jnp.int32),
        pltpu.VMEM((16,), jnp.int32),
    ],
)
def _bucket_kernel(src_hbm, dst_hbm, bout, cout, sbuf, dbuf, flat, curs, kscr):
    w = _worker_id()
    base = w * _EP
    pltpu.sync_copy(src_hbm.at[pl.ds(base, _EP)], sbuf)
    pltpu.sync_copy(dst_hbm.at[pl.ds(base, _EP)], dbuf)
    zeros16 = jnp.zeros((16,), jnp.int32)
    ones16 = jnp.ones((16,), jnp.int32)
    iota = lax.iota(jnp.int32, 16)
    for i in range(_NCH // 16):
        curs[pl.ds(i * 16, 16)] = zeros16

    def body(g, carry):
        dvec = dbuf[pl.ds(g * 16, 16)]
        svec = sbuf[pl.ds(g * 16, 16)]
        cid = jnp.bitwise_and(dvec, _NCH - 1)
        dl = jnp.right_shift(dvec, 7)
        packed = svec * 128 + dl
        skey, sval = plsc.sort_key_val(cid, packed)
        kscr[pl.ds(0, 16)] = skey
        prev = plsc.load_gather(kscr, [jnp.maximum(iota - 1, 0)])
        boundary = jnp.logical_or(iota == 0, skey != prev)
        segstart = plsc.cummax(jnp.where(boundary, iota, 0))
        rank = iota - segstart
        cur = plsc.load_gather(curs, [skey])
        pos = jnp.minimum(skey * _CAP + cur + rank, skey * _CAP + (_CAP - 1))
        plsc.store_scatter(flat, [pos], sval)
        plsc.addupdate_scatter(curs, [skey], ones16)
        return carry

    lax.fori_loop(0, _NG, body, 0)
    pltpu.sync_copy(flat, bout.at[w])
    pltpu.sync_copy(curs, cout.at[pl.ds(w * _NCH, _NCH)])


@functools.partial(
    pl.kernel,
    out_type=[jax.ShapeDtypeStruct((_CROWS, _NCH, _F), jnp.float32),
              jax.ShapeDtypeStruct((_CROWS, _NCH, _F), jnp.float32),
              jax.ShapeDtypeStruct((_CROWS, _NCH, _F), jnp.float32),
              jax.ShapeDtypeStruct((_CROWS, _NCH, _F), jnp.float32),
              jax.ShapeDtypeStruct((_NCH, _CROWS), jnp.float32)],
    mesh=_MESH,
    compiler_params=pltpu.CompilerParams(needs_layout_passes=False),
    scratch_types=[
        pltpu.VMEM((_CROWS, _F), jnp.float32),
        pltpu.VMEM((_CROWS, _F), jnp.float32),
        pltpu.VMEM((_CROWS, _F), jnp.float32),
        pltpu.VMEM((_CROWS, _F), jnp.float32),
        pltpu.VMEM((_CROWS,), jnp.int32),
        pltpu.VMEM((_CROWS,), jnp.float32),
        pltpu.VMEM((_CROWS,), jnp.float32),
        pltpu.VMEM((_CROWS,), jnp.int32),
        pltpu.VMEM((_CROWS,), jnp.int32),
        pltpu.VMEM((_CROWS,), jnp.int32),
        pltpu.VMEM((_NW * _CAP,), jnp.int32),
        pltpu.VMEM((_LCAP,), jnp.int32),
        pltpu.VMEM((_LCAP,), jnp.int32),
        pltpu.VMEM((_LCAP,), jnp.int32),
        pltpu.VMEM((_LCAP,), jnp.int32),
        pltpu.VMEM((2 * _RB, _F), jnp.float32),
        pltpu.VMEM((_CROWS, _F), jnp.float32),
        pltpu.VMEM((_NW * _NCH,), jnp.int32),
        pltpu.SemaphoreType.DMA,
    ],
)
def _accum_kernel(c_hbm, bkt_hbm, cnt_hbm, s1o, s2o, mno, mxo, cnto,
                  s1, s2, mn, mx, cli, clf, codd, segb, scur, sfirst, lists,
                  sidx, dlv, ssidx, ssdl, rows, fbuf, cstage, sem):
    w = _worker_id()
    pltpu.sync_copy(cnt_hbm, cstage)
    zeros16 = jnp.zeros((16,), jnp.float32)
    big16 = jnp.full((16,), 3e38, jnp.float32)
    zi16 = jnp.zeros((16,), jnp.int32)
    spare16 = jnp.full((16,), _SPARE, jnp.int32)
    ones16 = jnp.ones((16,), jnp.int32)
    iota = lax.iota(jnp.int32, 16)

    def chunk_body(chunk_i, carry0):
        c = w * _CPW + chunk_i

        def init_row(rr, carry):
            for j in range(8):
                sl = pl.ds(j * 16, 16)
                s1[rr, sl] = zeros16
                s2[rr, sl] = zeros16
                mn[rr, sl] = big16
                mx[rr, sl] = -big16
            return carry

        lax.fori_loop(0, _CROWS, init_row, 0)

        def zfill(k, carry):
            sl = pl.ds(k * 16, 16)
            sidx[sl] = zi16
            dlv[sl] = spare16
            ssidx[sl] = zi16
            ssdl[sl] = spare16
            return carry

        lax.fori_loop(0, _LCAP // 16, zfill, 0)
        for k in range(_CROWS // 16):
            cli[pl.ds(k * 16, 16)] = zi16

        descs = []
        for t in range(_NW):
            descs.append(pltpu.async_copy(
                bkt_hbm.at[t, pl.ds(c * _CAP, _CAP)],
                lists.at[pl.ds(t * _CAP, _CAP)], sem))
        for d in descs:
            d.wait()

        def compact_t(t, cur):
            nt = jnp.minimum(_sread(cstage, t * _NCH + c), _CAP)
            ng = (nt + 15) // 16

            def g_body(gi, cur2):
                v = lists[pl.ds(t * _CAP + gi * 16, 16)]
                msk = (gi * 16 + iota) < nt
                dl = jnp.bitwise_and(v, _NCH - 1)
                plsc.store_compressed(sidx.at[pl.ds(cur2, 16)],
                                      jnp.right_shift(v, 7), mask=msk)
                plsc.store_compressed(dlv.at[pl.ds(cur2, 16)], dl, mask=msk)
                plsc.addupdate_scatter(cli, [dl], ones16, mask=msk)
                return cur2 + jnp.sum(msk.astype(jnp.int32))

            return lax.fori_loop(0, ng, g_body, cur)

        m_total = lax.fori_loop(0, _NW, compact_t, 0)

        carry_s = 0
        for k in range(_CROWS // 16):
            sl = pl.ds(k * 16, 16)
            ck = cli[sl]
            ok = jnp.bitwise_and(ck, 1)
            codd[sl] = ok.astype(jnp.float32)
            ak = ck + ok
            incl = plsc.cumsum(ak) + carry_s
            ek = incl - ak
            segb[sl] = ek
            scur[sl] = ek
            carry_s = incl[15]
        total2 = jnp.minimum(carry_s, (_LCAP // _RB) * _RB)

        ngrp = (m_total + 15) // 16

        def sort_g(gi, carry):
            dl = dlv[pl.ds(gi * 16, 16)]
            sv = sidx[pl.ds(gi * 16, 16)]
            skey, sval = plsc.sort_key_val(dl, sv)
            kscr2 = lists
            kscr2[pl.ds(0, 16)] = skey
            prev = plsc.load_gather(kscr2, [jnp.maximum(iota - 1, 0)])
            boundary = jnp.logical_or(iota == 0, skey != prev)
            segstart = plsc.cummax(jnp.where(boundary, iota, 0))
            rank = iota - segstart
            cur = plsc.load_gather(scur, [skey])
            plsc.store_scatter(ssidx, [cur + rank], sval)
            plsc.store_scatter(ssdl, [cur + rank], skey)
            plsc.addupdate_scatter(scur, [skey], ones16)
            return carry

        lax.fori_loop(0, ngrp, sort_g, 0)

        for k in range(_CROWS // 16):
            sl = pl.ds(k * 16, 16)
            basev = segb[sl]
            kev = cli[sl]
            first = plsc.load_gather(ssidx, [basev])
            sfirst[sl] = first
            padmask = jnp.bitwise_and(kev, 1) == 1
            plsc.store_scatter(ssidx, [basev + kev], first, mask=padmask)

        pltpu.async_copy(c_hbm.at[sfirst], fbuf, sem)

        nb = (total2 + (_RB - 1)) // _RB

        def fire(b, par):
            pltpu.async_copy(
                c_hbm.at[ssidx.at[pl.ds(b * _RB, _RB)]],
                rows.at[pl.ds(par * _RB, _RB), :], sem)

        @pl.when(nb > 0)
        def _():
            fire(0, 0)

        pltpu.make_async_copy(c_hbm.at[sfirst], fbuf, sem).wait()

        def batch_body(b, carry):
            par = jnp.bitwise_and(b, 1)

            @pl.when(b + 1 < nb)
            def _():
                fire(b + 1, 1 - par)

            pltpu.make_async_copy(
                c_hbm.at[ssidx.at[pl.ds(b * _RB, _RB)]],
                rows.at[pl.ds(par * _RB, _RB), :], sem).wait()
            rbase = par * _RB

            def group_body(gi, carry2):
                ebase = gi * 32
                dla = ssdl[pl.ds(b * _RB + ebase, 16)]
                dlb = ssdl[pl.ds(b * _RB + ebase + 16, 16)]
                for half, dvec in ((0, dla), (1, dlb)):
                    for i in range(8):
                        d = dvec[2 * i]
                        q = rbase + ebase + half * 16 + 2 * i
                        for j in range(8):
                            sl = pl.ds(j * 16, 16)
                            g0 = rows[q, sl]
                            g1 = rows[q + 1, sl]
                            plsc.addupdate(s1.at[d, sl], g0 + g1)
                            plsc.addupdate(s2.at[d, sl], g0 * g0 + g1 * g1)
                            mn[d, sl] = jnp.minimum(mn[d, sl],
                                                    jnp.minimum(g0, g1))
                            mx[d, sl] = jnp.maximum(mx[d, sl],
                                                    jnp.maximum(g0, g1))
                return carry2

            lax.fori_loop(0, _RB // 32, group_body, 0)
            return carry

        lax.fori_loop(0, nb, batch_body, 0)

        def fix_body(rr, carry):
            pc = _sread(codd, rr)
            pcv = jnp.broadcast_to(pc, (16,))
            for j in range(8):
                sl = pl.ds(j * 16, 16)
                gf = fbuf[rr, sl]
                plsc.addupdate(s1.at[rr, sl], -(pcv * gf))
                plsc.addupdate(s2.at[rr, sl], -(pcv * gf * gf))
            return carry

        lax.fori_loop(0, _CROWS, fix_body, 0)

        for k in range(_CROWS // 16):
            sl = pl.ds(k * 16, 16)
            clf[sl] = cli[sl].astype(jnp.float32)

        pltpu.sync_copy(s1, s1o.at[:, c, :])
        pltpu.sync_copy(s2, s2o.at[:, c, :])
        pltpu.sync_copy(mn, mno.at[:, c, :])
        pltpu.sync_copy(mx, mxo.at[:, c, :])
        pltpu.sync_copy(clf, cnto.at[c])
        return carry0

    lax.fori_loop(0, _CPW, chunk_body, 0)


def kernel(x, edge_index, batch, W_pre_0, b_pre_0, W_post_0, b_post_0,
           W_lin_0, b_lin_0, W_pre_1, b_pre_1, W_post_1, b_post_1, W_lin_1,
           b_lin_1, W_mol1, b_mol1, W_mol2, b_mol2):
    src = edge_index[0]
    dst = edge_index[1]
    x_pad = jnp.pad(x, ((0, _NPAD - _N), (0, 0)))
    batch_pad = jnp.pad(batch, (0, _NPAD - _N), constant_values=_B).reshape(_NPAD, 1)
    W1_0, W2_0 = W_pre_0[:_F], W_pre_0[_F:]
    W1_1, W2_1 = W_pre_1[:_F], W_pre_1[_F:]
    r = lambda b: b.reshape(1, _F)

    A0, C0 = _t1_call(x_pad, W1_0, r(b_pre_0), W2_0)
    buckets, counts = _bucket_kernel(src, dst)

    S1a, S2a, Mna, Mxa, cnta = _accum_kernel(C0, buckets, counts)
    cnt2d = cnta.T.reshape(_NPAD, 1)
    h, A1, C1 = _combine_call(
        True, x_pad, A0, S1a.reshape(_NPAD, _F), S2a.reshape(_NPAD, _F),
        Mna.reshape(_NPAD, _F), Mxa.reshape(_NPAD, _F), cnt2d, W_post_0,
        r(b_post_0), W_lin_0, r(b_lin_0), (W1_1, r(b_pre_1), W2_1))

    S1b, S2b, Mnb, Mxb, _cntb = _accum_kernel(C1, buckets, counts)
    (h2,) = _combine_call(
        False, h, A1, S1b.reshape(_NPAD, _F), S2b.reshape(_NPAD, _F),
        Mnb.reshape(_NPAD, _F), Mxb.reshape(_NPAD, _F), cnt2d, W_post_1,
        r(b_post_1), W_lin_1, r(b_lin_1), ())

    return _pool_call(h2, batch_pad, W_mol1, r(b_mol1), W_mol2, r(b_mol2))


# speedup vs baseline: 1.7846x; 1.0739x over previous
import functools

import jax
import jax.numpy as jnp
import numpy as np
from jax import lax
from jax.experimental import pallas as pl
from jax.experimental.pallas import tpu as pltpu
from jax.experimental.pallas import tpu_sc as plsc

_N = 10000
_E = 320000
_F = 128
_B = 64

_NCH = 128
_CROWS = 80
_NPAD = _NCH * _CROWS
_NW = 32
_CPW = _NCH // _NW
_EP = _E // _NW
_NG = _EP // 16
_CAP = 256
_LCAP = 2816
_RB = 128
_SPARE = _CROWS - 1

_DEG_HIST = np.array([0]*28 + [200,600,1200,1800,2400,1800,1200,600,200] + [0]*28, dtype=np.float64)
_bins = np.arange(_DEG_HIST.shape[0], dtype=np.float64)
_AVG_DEG_LOG = float((np.log(_bins + 1.0) * _DEG_HIST).sum() / _DEG_HIST.sum())

_PREC = jax.lax.Precision.DEFAULT


def _dot(a, b):
    return jnp.dot(a, b, precision=_PREC, preferred_element_type=jnp.float32)


_RBLK = 1024
_NBLK = _NPAD // _RBLK


def _t1_body(x_ref, w1_ref, b1_ref, w2_ref, a_ref, c_ref):
    xb = x_ref[...]
    a_ref[...] = _dot(xb, w1_ref[...]) + b1_ref[...]
    c_ref[...] = _dot(xb, w2_ref[...])


def _t1_call(x_pad, W1, b1r, W2):
    full = lambda shp: pl.BlockSpec(shp, lambda i: (0, 0))
    row = pl.BlockSpec((_RBLK, _F), lambda i: (i, 0))
    return pl.pallas_call(
        _t1_body,
        grid=(_NBLK,),
        in_specs=[row, full((_F, _F)), full((1, _F)), full((_F, _F))],
        out_specs=[row, row],
        out_shape=[jax.ShapeDtypeStruct((_NPAD, _F), jnp.float32)] * 2,
    )(x_pad, W1, b1r, W2)


def _combine_body(layer0, x_ref, a_ref, s1_ref, s2_ref, mn_ref, mx_ref,
                  cnt_ref, wp_ref, bp_ref, wl_ref, bl_ref, *rest):
    if layer0:
        w1n_ref, b1n_ref, w2n_ref, h_ref, an_ref, cn_ref = rest
    else:
        (h_ref,) = rest
    xb = x_ref[...]
    A = a_ref[...]
    S1 = s1_ref[...]
    S2 = s2_ref[...]
    cnt = cnt_ref[...]
    has = cnt > 0.0
    cntc = jnp.maximum(cnt, 1.0)
    inv = 1.0 / cntc
    zero = jnp.zeros_like(A)
    mean = jnp.where(has, A + S1 * inv, zero)
    mn = jnp.where(has, A + mn_ref[...], zero)
    mx = jnp.where(has, A + mx_ref[...], zero)
    mean2 = jnp.where(has, A * A + (2.0 * A * S1 + S2) * inv, zero)
    std = jnp.sqrt(jnp.maximum(mean2 - mean * mean, 0.0) + 1e-5)
    agg = jnp.concatenate([mean, mn, mx, std], axis=-1)
    lg = jnp.log(cntc + 1.0)
    amp = lg * (1.0 / _AVG_DEG_LOG)
    att = _AVG_DEG_LOG / lg
    wp = wp_ref[...]
    out = (_dot(xb, wp[0:_F])
           + _dot(agg, wp[_F:5 * _F])
           + _dot(agg * amp, wp[5 * _F:9 * _F])
           + _dot(agg * att, wp[9 * _F:13 * _F])
           + bp_ref[...])
    out = _dot(out, wl_ref[...]) + bl_ref[...]
    if layer0:
        h = jnp.maximum(out, 0.0)
        h_ref[...] = h
        an_ref[...] = _dot(h, w1n_ref[...]) + b1n_ref[...]
        cn_ref[...] = _dot(h, w2n_ref[...])
    else:
        h_ref[...] = out


def _combine_call(layer0, x_pad, A, S1, S2, Mn, Mx, cnt2d, W_post, b_postr,
                  W_lin, b_linr, extra):
    full = lambda shp: pl.BlockSpec(shp, lambda i: (0, 0))
    row = pl.BlockSpec((_RBLK, _F), lambda i: (i, 0))
    col = pl.BlockSpec((_RBLK, 1), lambda i: (i, 0))
    in_specs = [row, row, row, row, row, row, col,
                full((13 * _F, _F)), full((1, _F)), full((_F, _F)), full((1, _F))]
    args = [x_pad, A, S1, S2, Mn, Mx, cnt2d, W_post, b_postr, W_lin, b_linr]
    if layer0:
        in_specs += [full((_F, _F)), full((1, _F)), full((_F, _F))]
        args += list(extra)
        out_specs = [row, row, row]
        out_shape = [jax.ShapeDtypeStruct((_NPAD, _F), jnp.float32)] * 3
    else:
        out_specs = [row]
        out_shape = [jax.ShapeDtypeStruct((_NPAD, _F), jnp.float32)]
    return pl.pallas_call(
        functools.partial(_combine_body, layer0),
        grid=(_NBLK,),
        in_specs=in_specs,
        out_specs=out_specs,
        out_shape=out_shape,
    )(*args)


def _pool_body(h_ref, b_ref, wm1_ref, bm1_ref, wm2_ref, bm2_ref, o_ref,
               psum, pcnt):
    i = pl.program_id(0)

    @pl.when(i == 0)
    def _():
        psum[...] = jnp.zeros_like(psum)
        pcnt[...] = jnp.zeros_like(pcnt)

    hb = h_ref[...]
    bb = b_ref[...]
    onehot = (bb == lax.broadcasted_iota(jnp.int32, (_RBLK, _B), 1)).astype(jnp.float32)
    psum[...] += lax.dot_general(onehot, hb, (((0,), (0,)), ((), ())),
                                 precision=_PREC, preferred_element_type=jnp.float32)
    pcnt[...] += jnp.broadcast_to(jnp.sum(onehot, axis=0)[:, None], (_B, _F))

    @pl.when(i == _NBLK - 1)
    def _():
        pooled = psum[...] / jnp.maximum(pcnt[...], 1.0)
        hmid = jnp.maximum(_dot(pooled, wm1_ref[...]) + bm1_ref[...], 0.0)
        o_ref[...] = _dot(hmid, wm2_ref[...]) + bm2_ref[...]


def _pool_call(h2, batch_pad, W_mol1, b_mol1r, W_mol2, b_mol2r):
    full = lambda shp: pl.BlockSpec(shp, lambda i: (0, 0))
    row = pl.BlockSpec((_RBLK, _F), lambda i: (i, 0))
    col = pl.BlockSpec((_RBLK, 1), lambda i: (i, 0))
    return pl.pallas_call(
        _pool_body,
        grid=(_NBLK,),
        in_specs=[row, col, full((_F, _F)), full((1, _F)), full((_F, _F)),
                  full((1, _F))],
        out_specs=pl.BlockSpec((_B, _F), lambda i: (0, 0)),
        out_shape=jax.ShapeDtypeStruct((_B, _F), jnp.float32),
        scratch_shapes=[pltpu.VMEM((_B, _F), jnp.float32),
                        pltpu.VMEM((_B, _F), jnp.float32)],
    )(h2, batch_pad, W_mol1, b_mol1r, W_mol2, b_mol2r)


_MESH = plsc.VectorSubcoreMesh(core_axis_name="c", subcore_axis_name="s")


def _worker_id():
    return lax.axis_index("s") * 2 + lax.axis_index("c")


def _sread(ref1d, idx):
    return plsc.load_gather(ref1d, [jnp.broadcast_to(idx, (16,))])[0]


@functools.partial(
    pl.kernel,
    out_type=[jax.ShapeDtypeStruct((_NW, _NCH * _CAP), jnp.int32),
              jax.ShapeDtypeStruct((_NW * _NCH,), jnp.int32)],
    mesh=_MESH,
    compiler_params=pltpu.CompilerParams(needs_layout_passes=False),
    scratch_types=[
        pltpu.VMEM((_EP,), jnp.int32),
        pltpu.VMEM((_EP,), jnp.int32),
        pltpu.VMEM((_NCH * _CAP,), jnp.int32),
        pltpu.VMEM((_NCH,), jnp.int32),
        pltpu.VMEM((16,), jnp.int32),
    ],
)
def _bucket_kernel(src_hbm, dst_hbm, bout, cout, sbuf, dbuf, flat, curs, kscr):
    w = _worker_id()
    base = w * _EP
    pltpu.sync_copy(src_hbm.at[pl.ds(base, _EP)], sbuf)
    pltpu.sync_copy(dst_hbm.at[pl.ds(base, _EP)], dbuf)
    zeros16 = jnp.zeros((16,), jnp.int32)
    ones16 = jnp.ones((16,), jnp.int32)
    iota = lax.iota(jnp.int32, 16)
    for i in range(_NCH // 16):
        curs[pl.ds(i * 16, 16)] = zeros16

    def body(g, carry):
        dvec = dbuf[pl.ds(g * 16, 16)]
        svec = sbuf[pl.ds(g * 16, 16)]
        cid = jnp.bitwise_and(dvec, _NCH - 1)
        dl = jnp.right_shift(dvec, 7)
        packed = svec * 128 + dl
        skey, sval = plsc.sort_key_val(cid, packed)
        kscr[pl.ds(0, 16)] = skey
        prev = plsc.load_gather(kscr, [jnp.maximum(iota - 1, 0)])
        boundary = jnp.logical_or(iota == 0, skey != prev)
        segstart = plsc.cummax(jnp.where(boundary, iota, 0))
        rank = iota - segstart
        cur = plsc.load_gather(curs, [skey])
        pos = jnp.minimum(skey * _CAP + cur + rank, skey * _CAP + (_CAP - 1))
        plsc.store_scatter(flat, [pos], sval)
        plsc.addupdate_scatter(curs, [skey], ones16)
        return carry

    lax.fori_loop(0, _NG, body, 0)
    pltpu.sync_copy(flat, bout.at[w])
    pltpu.sync_copy(curs, cout.at[pl.ds(w * _NCH, _NCH)])


@functools.partial(
    pl.kernel,
    out_type=[jax.ShapeDtypeStruct((_CROWS, _NCH, _F), jnp.float32),
              jax.ShapeDtypeStruct((_CROWS, _NCH, _F), jnp.float32),
              jax.ShapeDtypeStruct((_CROWS, _NCH, _F), jnp.float32),
              jax.ShapeDtypeStruct((_CROWS, _NCH, _F), jnp.float32),
              jax.ShapeDtypeStruct((_NCH, _CROWS), jnp.float32)],
    mesh=_MESH,
    compiler_params=pltpu.CompilerParams(needs_layout_passes=False),
    scratch_types=[
        pltpu.VMEM((_CROWS, _F), jnp.float32),
        pltpu.VMEM((_CROWS, _F), jnp.float32),
        pltpu.VMEM((_CROWS, _F), jnp.float32),
        pltpu.VMEM((_CROWS, _F), jnp.float32),
        pltpu.VMEM((_CROWS,), jnp.int32),
        pltpu.VMEM((_CROWS,), jnp.float32),
        pltpu.VMEM((_CROWS,), jnp.float32),
        pltpu.VMEM((_CROWS,), jnp.int32),
        pltpu.VMEM((_CROWS,), jnp.int32),
        pltpu.VMEM((_CROWS,), jnp.int32),
        pltpu.VMEM((_NW * _CAP,), jnp.int32),
        pltpu.VMEM((_LCAP,), jnp.int32),
        pltpu.VMEM((_LCAP,), jnp.int32),
        pltpu.VMEM((_LCAP,), jnp.int32),
        pltpu.VMEM((_LCAP,), jnp.int32),
        pltpu.VMEM((2 * _RB, _F), jnp.float32),
        pltpu.VMEM((_CROWS, _F), jnp.float32),
        pltpu.VMEM((_NW * _NCH,), jnp.int32),
        pltpu.SemaphoreType.DMA,
    ],
)
def _accum_kernel(c_hbm, bkt_hbm, cnt_hbm, s1o, s2o, mno, mxo, cnto,
                  s1, s2, mn, mx, cli, clf, codd, segb, scur, sfirst, lists,
                  sidx, dlv, ssidx, ssdl, rows, fbuf, cstage, sem):
    w = _worker_id()
    pltpu.sync_copy(cnt_hbm, cstage)
    zeros16 = jnp.zeros((16,), jnp.float32)
    big16 = jnp.full((16,), 3e38, jnp.float32)
    zi16 = jnp.zeros((16,), jnp.int32)
    spare16 = jnp.full((16,), _SPARE, jnp.int32)
    ones16 = jnp.ones((16,), jnp.int32)
    iota = lax.iota(jnp.int32, 16)

    def chunk_body(chunk_i, carry0):
        c = w * _CPW + chunk_i

        def init_row(rr, carry):
            for j in range(8):
                sl = pl.ds(j * 16, 16)
                s1[rr, sl] = zeros16
                s2[rr, sl] = zeros16
                mn[rr, sl] = big16
                mx[rr, sl] = -big16
            return carry

        lax.fori_loop(0, _CROWS, init_row, 0)

        def zfill(k, carry):
            sl = pl.ds(k * 16, 16)
            sidx[sl] = zi16
            dlv[sl] = spare16
            ssidx[sl] = zi16
            ssdl[sl] = spare16
            return carry

        lax.fori_loop(0, _LCAP // 16, zfill, 0)
        for k in range(_CROWS // 16):
            cli[pl.ds(k * 16, 16)] = zi16

        descs = []
        for t in range(_NW):
            descs.append(pltpu.async_copy(
                bkt_hbm.at[t, pl.ds(c * _CAP, _CAP)],
                lists.at[pl.ds(t * _CAP, _CAP)], sem))
        for d in descs:
            d.wait()

        def compact_t(t, cur):
            nt = jnp.minimum(_sread(cstage, t * _NCH + c), _CAP)
            ng = (nt + 15) // 16

            def g_body(gi, cur2):
                v = lists[pl.ds(t * _CAP + gi * 16, 16)]
                msk = (gi * 16 + iota) < nt
                dl = jnp.bitwise_and(v, _NCH - 1)
                plsc.store_compressed(sidx.at[pl.ds(cur2, 16)],
                                      jnp.right_shift(v, 7), mask=msk)
                plsc.store_compressed(dlv.at[pl.ds(cur2, 16)], dl, mask=msk)
                plsc.addupdate_scatter(cli, [dl], ones16, mask=msk)
                return cur2 + jnp.sum(msk.astype(jnp.int32))

            return lax.fori_loop(0, ng, g_body, cur)

        m_total = lax.fori_loop(0, _NW, compact_t, 0)

        carry_s = 0
        for k in range(_CROWS // 16):
            sl = pl.ds(k * 16, 16)
            ck = cli[sl]
            ok = jnp.bitwise_and(ck, 1)
            codd[sl] = ok.astype(jnp.float32)
            ak = ck + ok
            incl = plsc.cumsum(ak) + carry_s
            ek = incl - ak
            segb[sl] = ek
            scur[sl] = ek
            carry_s = incl[15]
        total2 = jnp.minimum(carry_s, (_LCAP // _RB) * _RB)

        ngrp = (m_total + 15) // 16

        def sort_g(gi, carry):
            dl = dlv[pl.ds(gi * 16, 16)]
            sv = sidx[pl.ds(gi * 16, 16)]
            skey, sval = plsc.sort_key_val(dl, sv)
            kscr2 = lists
            kscr2[pl.ds(0, 16)] = skey
            prev = plsc.load_gather(kscr2, [jnp.maximum(iota - 1, 0)])
            boundary = jnp.logical_or(iota == 0, skey != prev)
            segstart = plsc.cummax(jnp.where(boundary, iota, 0))
            rank = iota - segstart
            cur = plsc.load_gather(scur, [skey])
            plsc.store_scatter(ssidx, [cur + rank], sval)
            plsc.store_scatter(ssdl, [cur + rank], skey)
            plsc.addupdate_scatter(scur, [skey], ones16)
            return carry

        lax.fori_loop(0, ngrp, sort_g, 0)

        for k in range(_CROWS // 16):
            sl = pl.ds(k * 16, 16)
            basev = segb[sl]
            kev = cli[sl]
            first = plsc.load_gather(ssidx, [basev])
            sfirst[sl] = first
            padmask = jnp.bitwise_and(kev, 1) == 1
            plsc.store_scatter(ssidx, [basev + kev], first, mask=padmask)

        pltpu.async_copy(c_hbm.at[sfirst], fbuf, sem)

        nb = (total2 + (_RB - 1)) // _RB

        def fire(b, par):
            pltpu.async_copy(
                c_hbm.at[ssidx.at[pl.ds(b * _RB, _RB)]],
                rows.at[pl.ds(par * _RB, _RB), :], sem)

        @pl.when(nb > 0)
        def _():
            fire(0, 0)

        pltpu.make_async_copy(c_hbm.at[sfirst], fbuf, sem).wait()

        def batch_body(b, carry):
            par = jnp.bitwise_and(b, 1)

            @pl.when(b + 1 < nb)
            def _():
                fire(b + 1, 1 - par)

            pltpu.make_async_copy(
                c_hbm.at[ssidx.at[pl.ds(b * _RB, _RB)]],
                rows.at[pl.ds(par * _RB, _RB), :], sem).wait()
            rbase = par * _RB

            def group_body(gi, carry2):
                ebase = gi * 32
                dla = ssdl[pl.ds(b * _RB + ebase, 16)]
                dlb = ssdl[pl.ds(b * _RB + ebase + 16, 16)]
                for half, dvec in ((0, dla), (1, dlb)):
                    for i in range(8):
                        d = dvec[2 * i]
                        q = rbase + ebase + half * 16 + 2 * i
                        for j in range(8):
                            sl = pl.ds(j * 16, 16)
                            g0 = rows[q, sl]
                            g1 = rows[q + 1, sl]
                            plsc.addupdate(s1.at[d, sl], g0 + g1)
                            plsc.addupdate(s2.at[d, sl], g0 * g0 + g1 * g1)
                            mn[d, sl] = jnp.minimum(mn[d, sl],
                                                    jnp.minimum(g0, g1))
                            mx[d, sl] = jnp.maximum(mx[d, sl],
                                                    jnp.maximum(g0, g1))
                return carry2

            lax.fori_loop(0, _RB // 32, group_body, 0)
            return carry

        lax.fori_loop(0, nb, batch_body, 0)

        def fix_body(rr, carry):
            pc = _sread(codd, rr)
            pcv = jnp.broadcast_to(pc, (16,))
            for j in range(8):
                sl = pl.ds(j * 16, 16)
                gf = fbuf[rr, sl]
                plsc.addupdate(s1.at[rr, sl], -(pcv * gf))
                plsc.addupdate(s2.at[rr, sl], -(pcv * gf * gf))
            return carry

        lax.fori_loop(0, _CROWS, fix_body, 0)

        for k in range(_CROWS // 16):
            sl = pl.ds(k * 16, 16)
            clf[sl] = cli[sl].astype(jnp.float32)

        pltpu.sync_copy(s1, s1o.at[:, c, :])
        pltpu.sync_copy(s2, s2o.at[:, c, :])
        pltpu.sync_copy(mn, mno.at[:, c, :])
        pltpu.sync_copy(mx, mxo.at[:, c, :])
        pltpu.sync_copy(clf, cnto.at[c])
        return carry0

    lax.fori_loop(0, _CPW, chunk_body, 0)


def kernel(x, edge_index, batch, W_pre_0, b_pre_0, W_post_0, b_post_0,
           W_lin_0, b_lin_0, W_pre_1, b_pre_1, W_post_1, b_post_1, W_lin_1,
           b_lin_1, W_mol1, b_mol1, W_mol2, b_mol2):
    src = edge_index[0]
    dst = edge_index[1]
    x_pad = jnp.pad(x, ((0, _NPAD - _N), (0, 0)))
    batch_pad = jnp.pad(batch, (0, _NPAD - _N), constant_values=_B).reshape(_NPAD, 1)
    W1_0, W2_0 = W_pre_0[:_F], W_pre_0[_F:]
    W1_1, W2_1 = W_pre_1[:_F], W_pre_1[_F:]
    r = lambda b: b.reshape(1, _F)

    A0, C0 = _t1_call(x_pad, W1_0, r(b_pre_0), W2_0)
    buckets, counts = _bucket_kernel(src, dst)

    S1a, S2a, Mna, Mxa, cnta = _accum_kernel(C0, buckets, counts)
    cnt2d = cnta.T.reshape(_NPAD, 1)
    h, A1, C1 = _combine_call(
        True, x_pad, A0, S1a.reshape(_NPAD, _F), S2a.reshape(_NPAD, _F),
        Mna.reshape(_NPAD, _F), Mxa.reshape(_NPAD, _F), cnt2d, W_post_0,
        r(b_post_0), W_lin_0, r(b_lin_0), (W1_1, r(b_pre_1), W2_1))

    S1b, S2b, Mnb, Mxb, _cntb = _accum_kernel(C1, buckets, counts)
    (h2,) = _combine_call(
        False, h, A1, S1b.reshape(_NPAD, _F), S2b.reshape(_NPAD, _F),
        Mnb.reshape(_NPAD, _F), Mxb.reshape(_NPAD, _F), cnt2d, W_post_1,
        r(b_post_1), W_lin_1, r(b_lin_1), ())

    return _pool_call(h2, batch_pad, W_mol1, r(b_mol1), W_mol2, r(b_mol2))
